# baseline (speedup 1.0000x reference)
import functools

import jax
import jax.numpy as jnp
from jax import lax
from jax.experimental import pallas as pl
from jax.experimental.pallas import tpu as pltpu
from jax.experimental.pallas import tpu_sc as plsc

_N = 10000
_E = 320000
_F = 128
_H = 256
_C = 40

_NC = 2
_NS = 16
_CHUNK = 120
_EPAD = 322560
_ECHUNKS_AGG = _EPAD // (_NS * _CHUNK)
_EPW_AGG = _EPAD // _NS
_ECHUNKS_DEG = _EPAD // (_NC * _NS * _CHUNK)
_EPW_DEG = _EPAD // (_NC * _NS)
_NP = 10240
_NR = _NP
_NB_DEG = 10240
_ROWS_PER_SUB = _NP // _NS
_INIT_CHUNK = 128

_BN = 1024
_NBLK = _NP // _BN


_sc_mesh = plsc.VectorSubcoreMesh(core_axis_name="c", subcore_axis_name="s")


@functools.partial(
    pl.kernel,
    out_type=jax.ShapeDtypeStruct((_NC * _NB_DEG,), jnp.float32),
    mesh=_sc_mesh,
    scratch_types=[
        pltpu.VMEM((_CHUNK,), jnp.int32),
        pltpu.VMEM((128,), jnp.float32),
        pltpu.VMEM((640,), jnp.float32),
        pltpu.VMEM_SHARED((_NB_DEG,), jnp.float32),
    ],
)
def _deg(dst_hbm, out_hbm, dstb, onesb, zb, acc):
    c = lax.axis_index("c")
    s = lax.axis_index("s")
    w = c * _NS + s
    for i in range(640 // 16):
        zb[pl.ds(i * 16, 16)] = jnp.zeros((16,), jnp.float32)
    for i in range(128 // 16):
        onesb[pl.ds(i * 16, 16)] = jnp.ones((16,), jnp.float32)
    pltpu.sync_copy(zb, acc.at[pl.ds(s * 640, 640)])
    plsc.subcore_barrier()

    ebase = w * _EPW_DEG

    def body(j, carry):
        off = pl.multiple_of(ebase + j * _CHUNK, _CHUNK)
        pltpu.sync_copy(dst_hbm.at[pl.ds(off, _CHUNK)], dstb)
        pltpu.sync_copy(onesb.at[pl.ds(0, _CHUNK)], acc.at[dstb], add=True)
        return carry

    lax.fori_loop(0, _ECHUNKS_DEG, body, 0)
    plsc.subcore_barrier()
    pltpu.sync_copy(acc.at[pl.ds(s * 640, 640)], zb)
    pltpu.sync_copy(zb, out_hbm.at[pl.ds(c * _NB_DEG + s * 640, 640)])


@functools.partial(
    pl.kernel,
    out_type=jax.ShapeDtypeStruct((2 * _NP, _F), jnp.float32),
    mesh=_sc_mesh,
    scratch_types=[
        pltpu.VMEM((6, _CHUNK), jnp.int32),
        pltpu.VMEM((6, _CHUNK), jnp.int32),
        pltpu.VMEM((_CHUNK, _F), jnp.float32),
        pltpu.VMEM((_CHUNK, _F), jnp.float32),
        pltpu.VMEM((_CHUNK, _F), jnp.float32),
        pltpu.VMEM_SHARED((_NR, _F), jnp.float32),
        pltpu.SemaphoreType.DMA,
        pltpu.SemaphoreType.DMA,
        pltpu.SemaphoreType.DMA,
        pltpu.SemaphoreType.DMA,
        pltpu.SemaphoreType.DMA,
        pltpu.SemaphoreType.DMA,
        pltpu.SemaphoreType.DMA,
        pltpu.SemaphoreType.DMA,
        pltpu.SemaphoreType.DMA,
        pltpu.SemaphoreType.DMA,
        pltpu.SemaphoreType.DMA,
        pltpu.SemaphoreType.DMA,
    ],
)
def _agg(g_hbm, src_hbm, dst_hbm, out_hbm, sbs, dbs, rows0, rows1, rows2,
         acc, i0, i1, i2, i3, i4, i5, r0s, r1s, r2s, s0s, s1s, s2s):
    c = lax.axis_index("c")
    s = lax.axis_index("s")
    rbase = s * _ROWS_PER_SUB
    for k in range(5):
        r0 = rbase + k * _CHUNK
        pltpu.sync_copy(g_hbm.at[pl.ds(c * _NP + r0, _CHUNK)], rows0)
        pltpu.sync_copy(rows0, acc.at[pl.ds(r0, _CHUNK)])
    r0 = rbase + 5 * _CHUNK
    pltpu.sync_copy(g_hbm.at[pl.ds(c * _NP + r0, 40)], rows0.at[pl.ds(0, 40)])
    pltpu.sync_copy(rows0.at[pl.ds(0, 40)], acc.at[pl.ds(r0, 40)])
    plsc.subcore_barrier()

    sbase = c * _EPAD + s * _EPW_AGG
    dbase = s * _EPW_AGG
    isem = (i0, i1, i2, i3, i4, i5)
    rsem = (r0s, r1s, r2s)
    ssem = (s0s, s1s, s2s)
    rows = (rows0, rows1, rows2)

    def iload(j, q):
        off = pl.multiple_of(j * _CHUNK, _CHUNK)
        pltpu.async_copy(src_hbm.at[pl.ds(sbase + off, _CHUNK)],
                         sbs.at[q], isem[q])
        pltpu.async_copy(dst_hbm.at[pl.ds(dbase + off, _CHUNK)],
                         dbs.at[q], isem[q])

    def iwait(q):
        pltpu.make_async_copy(src_hbm.at[pl.ds(0, _CHUNK)], sbs.at[q],
                              isem[q]).wait()
        pltpu.make_async_copy(dst_hbm.at[pl.ds(0, _CHUNK)], dbs.at[q],
                              isem[q]).wait()

    def emit(j, q, b, first, has_next, do_iload):
        bn = (b + 1) % 3
        pltpu.make_async_copy(g_hbm.at[pl.ds(0, _CHUNK)], rows[b],
                              rsem[b]).wait()
        pltpu.async_copy(rows[b], acc.at[dbs.at[q]], ssem[b], add=True)
        if has_next:
            if not first:
                pltpu.make_async_copy(rows[bn], acc.at[dbs.at[q]],
                                      ssem[bn]).wait()
            qn = (q + 1) % 6
            iwait(qn)
            pltpu.async_copy(g_hbm.at[sbs.at[qn]], rows[bn], rsem[bn])
        if do_iload:
            iload(j + 3, (q + 3) % 6)

    iload(0, 0)
    iload(1, 1)
    iload(2, 2)
    iwait(0)
    pltpu.async_copy(g_hbm.at[sbs.at[0]], rows0, r0s)

    emit(0, 0, 0, True, True, True)
    emit(1, 1, 1, True, True, True)
    emit(2, 2, 2, False, True, True)
    emit(3, 3, 0, False, True, True)
    emit(4, 4, 1, False, True, True)
    emit(5, 5, 2, False, True, True)

    def body(i, carry):
        j = 6 * i
        emit(j + 0, 0, 0, False, True, True)
        emit(j + 1, 1, 1, False, True, True)
        emit(j + 2, 2, 2, False, True, True)
        emit(j + 3, 3, 0, False, True, True)
        emit(j + 4, 4, 1, False, True, True)
        emit(j + 5, 5, 2, False, True, True)
        return carry

    lax.fori_loop(1, 27, body, 0)
    emit(162, 0, 0, False, True, True)
    emit(163, 1, 1, False, True, True)
    emit(164, 2, 2, False, True, True)
    emit(165, 3, 0, False, True, False)
    emit(166, 4, 1, False, True, False)
    emit(167, 5, 2, False, False, False)
    pltpu.make_async_copy(rows0, acc.at[dbs.at[0]], s0s).wait()
    pltpu.make_async_copy(rows1, acc.at[dbs.at[1]], s1s).wait()
    pltpu.make_async_copy(rows2, acc.at[dbs.at[2]], s2s).wait()
    plsc.subcore_barrier()
    for k in range(5):
        r0 = rbase + k * _CHUNK
        pltpu.sync_copy(acc.at[pl.ds(r0, _CHUNK)], rows0)
        pltpu.sync_copy(rows0, out_hbm.at[pl.ds(c * _NP + r0, _CHUNK)])
    r0 = rbase + 5 * _CHUNK
    pltpu.sync_copy(acc.at[pl.ds(r0, 40)], rows0.at[pl.ds(0, 40)])
    pltpu.sync_copy(rows0.at[pl.ds(0, 40)], out_hbm.at[pl.ds(c * _NP + r0, 40)])


def _dinv(d0, d1):
    return lax.rsqrt(d0 + d1 + 1.0)


def _rowmask(dinv):
    rid = (lax.broadcasted_iota(jnp.int32, dinv.shape, 0)
           + pl.program_id(1) * _BN)
    return jnp.where(rid < _N, dinv, 0.0)


def _k1_body(x_ref, w_ref, b_ref, d0_ref, d1_ref, out_ref):
    dinv = _dinv(d0_ref[...], d1_ref[...])
    h = jnp.dot(x_ref[...], w_ref[...], preferred_element_type=jnp.float32)
    out_ref[...] = (h + b_ref[0:1, :]) * _rowmask(dinv)


def _k23_body(s0_ref, s1_ref, wa_ref, wb_ref, b_ref, d0_ref, d1_ref, out_ref):
    dinv = _dinv(d0_ref[...], d1_ref[...])
    xa = jax.nn.relu(dinv * s0_ref[...])
    xb = jax.nn.relu(dinv * s1_ref[...])
    h = (jnp.dot(xa, wa_ref[...], preferred_element_type=jnp.float32)
         + jnp.dot(xb, wb_ref[...], preferred_element_type=jnp.float32))
    out_ref[...] = (h + b_ref[0:1, :]) * _rowmask(dinv)


def _kout_body(s1a, s1b, s2a, s2b, s3a, s3b, w_ref, b_ref, d0_ref, d1_ref,
               out_ref):
    dinv = _dinv(d0_ref[...], d1_ref[...])
    acc = jnp.broadcast_to(b_ref[0:1, :], out_ref.shape)
    for l, sref in enumerate((s1a, s1b, s2a, s2b, s3a, s3b)):
        xl = jax.nn.relu(dinv * sref[...])
        acc = acc + jnp.dot(xl, w_ref[pl.ds(l * _F, _F), :],
                            preferred_element_type=jnp.float32)
    out_ref[...] = acc


_row_spec = pl.BlockSpec((_BN, _F), lambda h, b: (b, 0))
_deg_spec = pl.BlockSpec((_BN, 1), lambda h, b: (b, 0))
_out2n_spec = pl.BlockSpec((_BN, _F), lambda h, b: (h * _NBLK + b, 0))

_k1 = pl.pallas_call(
    _k1_body,
    grid=(2, _NBLK),
    in_specs=[
        _row_spec,
        pl.BlockSpec((_F, _F), lambda h, b: (0, h)),
        pl.BlockSpec((8, _F), lambda h, b: (0, h)),
        _deg_spec,
        _deg_spec,
    ],
    out_specs=_out2n_spec,
    out_shape=jax.ShapeDtypeStruct((2 * _NP, _F), jnp.float32),
)

_k23 = pl.pallas_call(
    _k23_body,
    grid=(2, _NBLK),
    in_specs=[
        _row_spec,
        _row_spec,
        pl.BlockSpec((_F, _F), lambda h, b: (0, h)),
        pl.BlockSpec((_F, _F), lambda h, b: (0, h)),
        pl.BlockSpec((8, _F), lambda h, b: (0, h)),
        _deg_spec,
        _deg_spec,
    ],
    out_specs=_out2n_spec,
    out_shape=jax.ShapeDtypeStruct((2 * _NP, _F), jnp.float32),
)

_kout = pl.pallas_call(
    _kout_body,
    grid=(_NBLK,),
    in_specs=[pl.BlockSpec((_BN, _F), lambda b: (b, 0))] * 6
    + [
        pl.BlockSpec((6 * _F, _F), lambda b: (0, 0)),
        pl.BlockSpec((8, _F), lambda b: (0, 0)),
        pl.BlockSpec((_BN, 1), lambda b: (b, 0)),
        pl.BlockSpec((_BN, 1), lambda b: (b, 0)),
    ],
    out_specs=pl.BlockSpec((_BN, _F), lambda b: (b, 0)),
    out_shape=jax.ShapeDtypeStruct((_NP, _F), jnp.float32),
)


def kernel(x, edge_index, W1, b1, W2, b2, W3, b3, Wout, bout):
    src = edge_index[0]
    dst = edge_index[1]
    npad = _EPAD - _E
    fill = jnp.arange(npad, dtype=jnp.int32)
    padidx = _N + (fill % (_NP - _N))
    src_p = jnp.concatenate([src, padidx])
    dst_p = jnp.concatenate([dst, padidx])
    src_sh = jnp.concatenate([src_p, src_p + _NP])

    deg2 = _deg(dst_p)
    d0 = deg2[:_NP].reshape(_NP, 1)
    d1 = deg2[_NB_DEG:].reshape(_NP, 1)

    b1b = jnp.broadcast_to(b1, (8, _H))
    b2b = jnp.broadcast_to(b2, (8, _H))
    b3b = jnp.broadcast_to(b3, (8, _H))
    wout_p = jnp.pad(Wout, ((0, 0), (0, _F - _C)))
    bout_p = jnp.broadcast_to(jnp.pad(bout, (0, _F - _C)), (8, _F))

    x_p = jnp.pad(x, ((0, _NP - _N), (0, 0)))
    g1 = _k1(x_p, W1, b1b, d0, d1)
    s1 = _agg(g1, src_sh, dst_p)
    g2 = _k23(s1[:_NP], s1[_NP:], W2[:_F], W2[_F:], b2b, d0, d1)
    s2 = _agg(g2, src_sh, dst_p)
    g3 = _k23(s2[:_NP], s2[_NP:], W3[:_F], W3[_F:], b3b, d0, d1)
    s3 = _agg(g3, src_sh, dst_p)

    out = _kout(s1[:_NP], s1[_NP:], s2[:_NP], s2[_NP:], s3[:_NP], s3[_NP:],
                wout_p, bout_p, d0, d1)
    return out[:_N, :_C]


# speedup vs baseline: 1.0470x; 1.0470x over previous
import functools

import jax
import jax.numpy as jnp
from jax import lax
from jax.experimental import pallas as pl
from jax.experimental.pallas import tpu as pltpu
from jax.experimental.pallas import tpu_sc as plsc

_N = 10000
_E = 320000
_F = 128
_H = 256
_C = 40

_NC = 2
_NS = 16
_CHUNK = 128
_EPAD = 323584
_ECHUNKS_AGG = _EPAD // (_NS * _CHUNK)
_EPW_AGG = _EPAD // _NS
_ECHUNKS_DEG = _EPAD // (_NC * _NS * _CHUNK)
_EPW_DEG = _EPAD // (_NC * _NS)
_NP = 10240
_NR = _NP
_NB_DEG = 10240
_ROWS_PER_SUB = _NP // _NS
_INIT_CHUNK = 128

_BN = 1024
_NBLK = _NP // _BN


_sc_mesh = plsc.VectorSubcoreMesh(core_axis_name="c", subcore_axis_name="s")


@functools.partial(
    pl.kernel,
    out_type=jax.ShapeDtypeStruct((_NC * _NB_DEG,), jnp.float32),
    mesh=_sc_mesh,
    scratch_types=[
        pltpu.VMEM((_CHUNK,), jnp.int32),
        pltpu.VMEM((128,), jnp.float32),
        pltpu.VMEM((640,), jnp.float32),
        pltpu.VMEM_SHARED((_NB_DEG,), jnp.float32),
    ],
)
def _deg(dst_hbm, out_hbm, dstb, onesb, zb, acc):
    c = lax.axis_index("c")
    s = lax.axis_index("s")
    w = c * _NS + s
    for i in range(640 // 16):
        zb[pl.ds(i * 16, 16)] = jnp.zeros((16,), jnp.float32)
    for i in range(128 // 16):
        onesb[pl.ds(i * 16, 16)] = jnp.ones((16,), jnp.float32)
    pltpu.sync_copy(zb, acc.at[pl.ds(s * 640, 640)])
    plsc.subcore_barrier()

    ebase = w * _EPW_DEG

    def body(j, carry):
        off = pl.multiple_of(ebase + j * _CHUNK, _CHUNK)
        pltpu.sync_copy(dst_hbm.at[pl.ds(off, _CHUNK)], dstb)
        pltpu.sync_copy(onesb.at[pl.ds(0, _CHUNK)], acc.at[dstb], add=True)
        return carry

    lax.fori_loop(0, _ECHUNKS_DEG, body, 0)
    plsc.subcore_barrier()
    pltpu.sync_copy(acc.at[pl.ds(s * 640, 640)], zb)
    pltpu.sync_copy(zb, out_hbm.at[pl.ds(c * _NB_DEG + s * 640, 640)])


@functools.partial(
    pl.kernel,
    out_type=jax.ShapeDtypeStruct((2 * _NP, _F), jnp.float32),
    mesh=_sc_mesh,
    scratch_types=[
        pltpu.VMEM((4, _CHUNK), jnp.int32),
        pltpu.VMEM((4, _CHUNK), jnp.int32),
        pltpu.VMEM((_CHUNK, _F), jnp.float32),
        pltpu.VMEM((_CHUNK, _F), jnp.float32),
        pltpu.VMEM_SHARED((_NR, _F), jnp.float32),
        pltpu.SemaphoreType.DMA,
        pltpu.SemaphoreType.DMA,
        pltpu.SemaphoreType.DMA,
        pltpu.SemaphoreType.DMA,
        pltpu.SemaphoreType.DMA,
        pltpu.SemaphoreType.DMA,
        pltpu.SemaphoreType.DMA,
        pltpu.SemaphoreType.DMA,
    ],
)
def _agg(g_hbm, src_hbm, dst_hbm, out_hbm, sbs, dbs, rows0, rows1,
         acc, i0, i1, i2, i3, r0s, r1s, s0s, s1s):
    c = lax.axis_index("c")
    s = lax.axis_index("s")
    rbase = s * _ROWS_PER_SUB
    for k in range(_ROWS_PER_SUB // _INIT_CHUNK):
        r0 = rbase + k * _INIT_CHUNK
        pltpu.sync_copy(g_hbm.at[pl.ds(c * _NP + r0, _INIT_CHUNK)], rows0)
        pltpu.sync_copy(rows0, acc.at[pl.ds(r0, _INIT_CHUNK)])
    plsc.subcore_barrier()

    sbase = c * _EPAD + s * _EPW_AGG
    dbase = s * _EPW_AGG
    isem = (i0, i1, i2, i3)
    rsem = (r0s, r1s)
    ssem = (s0s, s1s)
    rows = (rows0, rows1)

    def iload(j, q):
        off = pl.multiple_of(j * _CHUNK, _CHUNK)
        pltpu.async_copy(src_hbm.at[pl.ds(sbase + off, _CHUNK)],
                         sbs.at[q], isem[q])
        pltpu.async_copy(dst_hbm.at[pl.ds(dbase + off, _CHUNK)],
                         dbs.at[q], isem[q])

    def iwait(q):
        pltpu.make_async_copy(src_hbm.at[pl.ds(0, _CHUNK)], sbs.at[q],
                              isem[q]).wait()
        pltpu.make_async_copy(dst_hbm.at[pl.ds(0, _CHUNK)], dbs.at[q],
                              isem[q]).wait()

    def emit(j, q, b, first, has_next, do_iload):
        pltpu.make_async_copy(g_hbm.at[pl.ds(0, _CHUNK)], rows[b],
                              rsem[b]).wait()
        pltpu.async_copy(rows[b], acc.at[dbs.at[q]], ssem[b], add=True)
        if has_next:
            if not first:
                pltpu.make_async_copy(rows[1 - b], acc.at[dbs.at[q]],
                                      ssem[1 - b]).wait()
            qn = (q + 1) % 4
            iwait(qn)
            pltpu.async_copy(g_hbm.at[sbs.at[qn]], rows[1 - b], rsem[1 - b])
        if do_iload:
            iload(j + 3, (q + 3) % 4)

    iload(0, 0)
    iload(1, 1)
    iload(2, 2)
    iwait(0)
    pltpu.async_copy(g_hbm.at[sbs.at[0]], rows0, r0s)

    emit(0, 0, 0, True, True, True)
    emit(1, 1, 1, False, True, True)
    emit(2, 2, 0, False, True, True)
    emit(3, 3, 1, False, True, True)

    def body(i, carry):
        j = 4 * i
        emit(j + 0, 0, 0, False, True, True)
        emit(j + 1, 1, 1, False, True, True)
        emit(j + 2, 2, 0, False, True, True)
        emit(j + 3, 3, 1, False, True, True)
        return carry

    lax.fori_loop(1, 38, body, 0)
    emit(152, 0, 0, False, True, True)
    emit(153, 1, 1, False, True, True)
    emit(154, 2, 0, False, True, True)
    emit(155, 3, 1, False, True, False)
    emit(156, 0, 0, False, True, False)
    emit(157, 1, 1, False, False, False)
    pltpu.make_async_copy(rows0, acc.at[dbs.at[0]], s0s).wait()
    pltpu.make_async_copy(rows1, acc.at[dbs.at[1]], s1s).wait()
    plsc.subcore_barrier()
    for k in range(_ROWS_PER_SUB // _INIT_CHUNK):
        r0 = rbase + k * _INIT_CHUNK
        pltpu.sync_copy(acc.at[pl.ds(r0, _INIT_CHUNK)], rows0)
        pltpu.sync_copy(rows0, out_hbm.at[pl.ds(c * _NP + r0, _INIT_CHUNK)])


def _dinv(d0, d1):
    return lax.rsqrt(d0 + d1 + 1.0)


def _rowmask(dinv):
    rid = (lax.broadcasted_iota(jnp.int32, dinv.shape, 0)
           + pl.program_id(1) * _BN)
    return jnp.where(rid < _N, dinv, 0.0)


def _k1_body(x_ref, w_ref, b_ref, d0_ref, d1_ref, out_ref):
    dinv = _dinv(d0_ref[...], d1_ref[...])
    h = jnp.dot(x_ref[...], w_ref[...], preferred_element_type=jnp.float32)
    out_ref[...] = (h + b_ref[0:1, :]) * _rowmask(dinv)


def _k23_body(s0_ref, s1_ref, wa_ref, wb_ref, b_ref, d0_ref, d1_ref, out_ref):
    dinv = _dinv(d0_ref[...], d1_ref[...])
    xa = jax.nn.relu(dinv * s0_ref[...])
    xb = jax.nn.relu(dinv * s1_ref[...])
    h = (jnp.dot(xa, wa_ref[...], preferred_element_type=jnp.float32)
         + jnp.dot(xb, wb_ref[...], preferred_element_type=jnp.float32))
    out_ref[...] = (h + b_ref[0:1, :]) * _rowmask(dinv)


def _kout_body(s1a, s1b, s2a, s2b, s3a, s3b, w_ref, b_ref, d0_ref, d1_ref,
               out_ref):
    dinv = _dinv(d0_ref[...], d1_ref[...])
    acc = jnp.broadcast_to(b_ref[0:1, :], out_ref.shape)
    for l, sref in enumerate((s1a, s1b, s2a, s2b, s3a, s3b)):
        xl = jax.nn.relu(dinv * sref[...])
        acc = acc + jnp.dot(xl, w_ref[pl.ds(l * _F, _F), :],
                            preferred_element_type=jnp.float32)
    out_ref[...] = acc


_row_spec = pl.BlockSpec((_BN, _F), lambda h, b: (b, 0))
_row_lo = pl.BlockSpec((_BN, _F), lambda h, b: (b, 0))
_row_hi = pl.BlockSpec((_BN, _F), lambda h, b: (_NBLK + b, 0))
_deg_lo = pl.BlockSpec((_BN, 1), lambda h, b: (b, 0))
_deg_hi = pl.BlockSpec((_BN, 1), lambda h, b: (_NBLK + b, 0))
_out2n_spec = pl.BlockSpec((_BN, _F), lambda h, b: (h * _NBLK + b, 0))

_k1 = pl.pallas_call(
    _k1_body,
    grid=(2, _NBLK),
    in_specs=[
        _row_spec,
        pl.BlockSpec((_F, _F), lambda h, b: (0, h)),
        pl.BlockSpec((8, _F), lambda h, b: (0, h)),
        _deg_lo,
        _deg_hi,
    ],
    out_specs=_out2n_spec,
    out_shape=jax.ShapeDtypeStruct((2 * _NP, _F), jnp.float32),
)

_k23 = pl.pallas_call(
    _k23_body,
    grid=(2, _NBLK),
    in_specs=[
        _row_lo,
        _row_hi,
        pl.BlockSpec((_F, _F), lambda h, b: (0, h)),
        pl.BlockSpec((_F, _F), lambda h, b: (1, h)),
        pl.BlockSpec((8, _F), lambda h, b: (0, h)),
        _deg_lo,
        _deg_hi,
    ],
    out_specs=_out2n_spec,
    out_shape=jax.ShapeDtypeStruct((2 * _NP, _F), jnp.float32),
)

_kout = pl.pallas_call(
    _kout_body,
    grid=(_NBLK,),
    in_specs=[
        pl.BlockSpec((_BN, _F), lambda b: (b, 0)),
        pl.BlockSpec((_BN, _F), lambda b: (_NBLK + b, 0)),
        pl.BlockSpec((_BN, _F), lambda b: (b, 0)),
        pl.BlockSpec((_BN, _F), lambda b: (_NBLK + b, 0)),
        pl.BlockSpec((_BN, _F), lambda b: (b, 0)),
        pl.BlockSpec((_BN, _F), lambda b: (_NBLK + b, 0)),
        pl.BlockSpec((6 * _F, _F), lambda b: (0, 0)),
        pl.BlockSpec((8, _F), lambda b: (0, 0)),
        pl.BlockSpec((_BN, 1), lambda b: (b, 0)),
        pl.BlockSpec((_BN, 1), lambda b: (_NBLK + b, 0)),
    ],
    out_specs=pl.BlockSpec((_BN, _F), lambda b: (b, 0)),
    out_shape=jax.ShapeDtypeStruct((_NP, _F), jnp.float32),
)


def kernel(x, edge_index, W1, b1, W2, b2, W3, b3, Wout, bout):
    src = edge_index[0]
    dst = edge_index[1]
    npad = _EPAD - _E
    fill = jnp.arange(npad, dtype=jnp.int32)
    padidx = _N + (fill % (_NP - _N))
    src_p = jnp.concatenate([src, padidx])
    dst_p = jnp.concatenate([dst, padidx])
    src_sh = jnp.concatenate([src_p, src_p + _NP])

    dd = _deg(dst_p).reshape(2 * _NB_DEG, 1)

    b1b = jnp.broadcast_to(b1, (8, _H))
    b2b = jnp.broadcast_to(b2, (8, _H))
    b3b = jnp.broadcast_to(b3, (8, _H))
    wout_p = jnp.pad(Wout, ((0, 0), (0, _F - _C)))
    bout_p = jnp.broadcast_to(jnp.pad(bout, (0, _F - _C)), (8, _F))

    x_p = jnp.pad(x, ((0, _NP - _N), (0, 0)))
    g1 = _k1(x_p, W1, b1b, dd, dd)
    s1 = _agg(g1, src_sh, dst_p)
    g2 = _k23(s1, s1, W2, W2, b2b, dd, dd)
    s2 = _agg(g2, src_sh, dst_p)
    g3 = _k23(s2, s2, W3, W3, b3b, dd, dd)
    s3 = _agg(g3, src_sh, dst_p)

    out = _kout(s1, s1, s2, s2, s3, s3, wout_p, bout_p, dd, dd)
    return out[:_N, :_C]


# speedup vs baseline: 1.0752x; 1.0269x over previous
import functools

import jax
import jax.numpy as jnp
from jax import lax
from jax.experimental import pallas as pl
from jax.experimental.pallas import tpu as pltpu
from jax.experimental.pallas import tpu_sc as plsc

_N = 10000
_E = 320000
_F = 128
_H = 256
_C = 40

_NC = 2
_NS = 16
_CHUNK = 128
_EPAD = 323584
_ECHUNKS_AGG = _EPAD // (_NS * _CHUNK)
_EPW_AGG = _EPAD // _NS
_ECHUNKS_DEG = _EPAD // (_NC * _NS * _CHUNK)
_EPW_DEG = _EPAD // (_NC * _NS)
_NP = 10240
_NR = _NP
_NB_DEG = 10240
_ROWS_PER_SUB = _NP // _NS
_INIT_CHUNK = 128

_BN = 1024
_NBLK = _NP // _BN


_sc_mesh = plsc.VectorSubcoreMesh(core_axis_name="c", subcore_axis_name="s")


@functools.partial(
    pl.kernel,
    out_type=jax.ShapeDtypeStruct((_NC * _NB_DEG,), jnp.float32),
    mesh=_sc_mesh,
    scratch_types=[
        pltpu.VMEM((4, _CHUNK), jnp.int32),
        pltpu.VMEM((128,), jnp.float32),
        pltpu.VMEM((640,), jnp.float32),
        pltpu.VMEM_SHARED((_NB_DEG,), jnp.float32),
        pltpu.SemaphoreType.DMA,
        pltpu.SemaphoreType.DMA,
        pltpu.SemaphoreType.DMA,
        pltpu.SemaphoreType.DMA,
        pltpu.SemaphoreType.DMA,
        pltpu.SemaphoreType.DMA,
        pltpu.SemaphoreType.DMA,
        pltpu.SemaphoreType.DMA,
    ],
)
def _deg(dst_hbm, out_hbm, dbs, onesb, zb, acc, i0, i1, i2, i3,
         s0, s1, s2, s3):
    c = lax.axis_index("c")
    s = lax.axis_index("s")
    w = c * _NS + s
    isem = (i0, i1, i2, i3)
    ssem = (s0, s1, s2, s3)
    ebase = w * _EPW_DEG
    ones = onesb.at[pl.ds(0, _CHUNK)]

    def iload(j, q):
        off = pl.multiple_of(ebase + j * _CHUNK, _CHUNK)
        pltpu.async_copy(dst_hbm.at[pl.ds(off, _CHUNK)], dbs.at[q], isem[q])

    iload(0, 0)
    iload(1, 1)
    for i in range(640 // 16):
        zb[pl.ds(i * 16, 16)] = jnp.zeros((16,), jnp.float32)
    for i in range(128 // 16):
        onesb[pl.ds(i * 16, 16)] = jnp.ones((16,), jnp.float32)
    pltpu.sync_copy(zb, acc.at[pl.ds(s * 640, 640)])
    plsc.subcore_barrier()

    def emit(j, q, do_swait, do_iload):
        pltpu.make_async_copy(dst_hbm.at[pl.ds(0, _CHUNK)], dbs.at[q],
                              isem[q]).wait()
        pltpu.async_copy(ones, acc.at[dbs.at[q]], ssem[q], add=True)
        if do_swait:
            q2 = (q + 2) % 4
            pltpu.make_async_copy(ones, acc.at[dbs.at[q2]], ssem[q2]).wait()
        if do_iload:
            iload(j + 2, (q + 2) % 4)

    emit(0, 0, False, True)
    emit(1, 1, False, True)
    emit(2, 2, True, True)
    emit(3, 3, True, True)

    def body(i, carry):
        j = 4 * i
        emit(j + 0, 0, True, True)
        emit(j + 1, 1, True, True)
        emit(j + 2, 2, True, True)
        emit(j + 3, 3, True, True)
        return carry

    lax.fori_loop(1, 18, body, 0)
    emit(72, 0, True, True)
    emit(73, 1, True, True)
    emit(74, 2, True, True)
    emit(75, 3, True, True)
    emit(76, 0, True, True)
    emit(77, 1, True, False)
    emit(78, 2, True, False)
    pltpu.make_async_copy(ones, acc.at[dbs.at[1]], ssem[1]).wait()
    pltpu.make_async_copy(ones, acc.at[dbs.at[2]], ssem[2]).wait()
    plsc.subcore_barrier()
    pltpu.sync_copy(acc.at[pl.ds(s * 640, 640)], zb)
    pltpu.sync_copy(zb, out_hbm.at[pl.ds(c * _NB_DEG + s * 640, 640)])


@functools.partial(
    pl.kernel,
    out_type=jax.ShapeDtypeStruct((2 * _NP, _F), jnp.float32),
    mesh=_sc_mesh,
    scratch_types=[
        pltpu.VMEM((4, _CHUNK), jnp.int32),
        pltpu.VMEM((4, _CHUNK), jnp.int32),
        pltpu.VMEM((_CHUNK, _F), jnp.float32),
        pltpu.VMEM((_CHUNK, _F), jnp.float32),
        pltpu.VMEM_SHARED((_NR, _F), jnp.float32),
        pltpu.SemaphoreType.DMA,
        pltpu.SemaphoreType.DMA,
        pltpu.SemaphoreType.DMA,
        pltpu.SemaphoreType.DMA,
        pltpu.SemaphoreType.DMA,
        pltpu.SemaphoreType.DMA,
        pltpu.SemaphoreType.DMA,
        pltpu.SemaphoreType.DMA,
    ],
)
def _agg(g_hbm, src_hbm, dst_hbm, out_hbm, sbs, dbs, rows0, rows1,
         acc, i0, i1, i2, i3, r0s, r1s, s0s, s1s):
    c = lax.axis_index("c")
    s = lax.axis_index("s")
    rbase = s * _ROWS_PER_SUB
    for k in range(_ROWS_PER_SUB // _INIT_CHUNK):
        r0 = rbase + k * _INIT_CHUNK
        pltpu.sync_copy(g_hbm.at[pl.ds(c * _NP + r0, _INIT_CHUNK)], rows0)
        pltpu.sync_copy(rows0, acc.at[pl.ds(r0, _INIT_CHUNK)])
    plsc.subcore_barrier()

    sbase = c * _EPAD + s * _EPW_AGG
    dbase = s * _EPW_AGG
    isem = (i0, i1, i2, i3)
    rsem = (r0s, r1s)
    ssem = (s0s, s1s)
    rows = (rows0, rows1)

    def iload(j, q):
        off = pl.multiple_of(j * _CHUNK, _CHUNK)
        pltpu.async_copy(src_hbm.at[pl.ds(sbase + off, _CHUNK)],
                         sbs.at[q], isem[q])
        pltpu.async_copy(dst_hbm.at[pl.ds(dbase + off, _CHUNK)],
                         dbs.at[q], isem[q])

    def iwait(q):
        pltpu.make_async_copy(src_hbm.at[pl.ds(0, _CHUNK)], sbs.at[q],
                              isem[q]).wait()
        pltpu.make_async_copy(dst_hbm.at[pl.ds(0, _CHUNK)], dbs.at[q],
                              isem[q]).wait()

    def emit(j, q, b, first, has_next, do_iload):
        pltpu.make_async_copy(g_hbm.at[pl.ds(0, _CHUNK)], rows[b],
                              rsem[b]).wait()
        pltpu.async_copy(rows[b], acc.at[dbs.at[q]], ssem[b], add=True)
        if has_next:
            if not first:
                pltpu.make_async_copy(rows[1 - b], acc.at[dbs.at[q]],
                                      ssem[1 - b]).wait()
            qn = (q + 1) % 4
            iwait(qn)
            pltpu.async_copy(g_hbm.at[sbs.at[qn]], rows[1 - b], rsem[1 - b])
        if do_iload:
            iload(j + 3, (q + 3) % 4)

    iload(0, 0)
    iload(1, 1)
    iload(2, 2)
    iwait(0)
    pltpu.async_copy(g_hbm.at[sbs.at[0]], rows0, r0s)

    emit(0, 0, 0, True, True, True)
    emit(1, 1, 1, False, True, True)
    emit(2, 2, 0, False, True, True)
    emit(3, 3, 1, False, True, True)

    def body(i, carry):
        j = 4 * i
        emit(j + 0, 0, 0, False, True, True)
        emit(j + 1, 1, 1, False, True, True)
        emit(j + 2, 2, 0, False, True, True)
        emit(j + 3, 3, 1, False, True, True)
        return carry

    lax.fori_loop(1, 38, body, 0)
    emit(152, 0, 0, False, True, True)
    emit(153, 1, 1, False, True, True)
    emit(154, 2, 0, False, True, True)
    emit(155, 3, 1, False, True, False)
    emit(156, 0, 0, False, True, False)
    emit(157, 1, 1, False, False, False)
    pltpu.make_async_copy(rows0, acc.at[dbs.at[0]], s0s).wait()
    pltpu.make_async_copy(rows1, acc.at[dbs.at[1]], s1s).wait()
    plsc.subcore_barrier()
    for k in range(_ROWS_PER_SUB // _INIT_CHUNK):
        r0 = rbase + k * _INIT_CHUNK
        pltpu.sync_copy(acc.at[pl.ds(r0, _INIT_CHUNK)], rows0)
        pltpu.sync_copy(rows0, out_hbm.at[pl.ds(c * _NP + r0, _INIT_CHUNK)])


def _dinv(d0, d1):
    return lax.rsqrt(d0 + d1 + 1.0)


def _rowmask(dinv):
    rid = (lax.broadcasted_iota(jnp.int32, dinv.shape, 0)
           + pl.program_id(1) * _BN)
    return jnp.where(rid < _N, dinv, 0.0)


def _k1_body(x_ref, w_ref, b_ref, d0_ref, d1_ref, out_ref):
    dinv = _dinv(d0_ref[...], d1_ref[...])
    h = jnp.dot(x_ref[...], w_ref[...], preferred_element_type=jnp.float32)
    out_ref[...] = (h + b_ref[0:1, :]) * _rowmask(dinv)


def _k23_body(s0_ref, s1_ref, wa_ref, wb_ref, b_ref, d0_ref, d1_ref, out_ref):
    dinv = _dinv(d0_ref[...], d1_ref[...])
    xa = jax.nn.relu(dinv * s0_ref[...])
    xb = jax.nn.relu(dinv * s1_ref[...])
    h = (jnp.dot(xa, wa_ref[...], preferred_element_type=jnp.float32)
         + jnp.dot(xb, wb_ref[...], preferred_element_type=jnp.float32))
    out_ref[...] = (h + b_ref[0:1, :]) * _rowmask(dinv)


def _kout_body(s1a, s1b, s2a, s2b, s3a, s3b, w_ref, b_ref, d0_ref, d1_ref,
               out_ref):
    dinv = _dinv(d0_ref[...], d1_ref[...])
    acc = jnp.broadcast_to(b_ref[0:1, :], out_ref.shape)
    for l, sref in enumerate((s1a, s1b, s2a, s2b, s3a, s3b)):
        xl = jax.nn.relu(dinv * sref[...])
        acc = acc + jnp.dot(xl, w_ref[pl.ds(l * _F, _F), :],
                            preferred_element_type=jnp.float32)
    out_ref[...] = acc


_row_spec = pl.BlockSpec((_BN, _F), lambda h, b: (b, 0))
_row_lo = pl.BlockSpec((_BN, _F), lambda h, b: (b, 0))
_row_hi = pl.BlockSpec((_BN, _F), lambda h, b: (_NBLK + b, 0))
_deg_lo = pl.BlockSpec((_BN, 1), lambda h, b: (b, 0))
_deg_hi = pl.BlockSpec((_BN, 1), lambda h, b: (_NBLK + b, 0))
_out2n_spec = pl.BlockSpec((_BN, _F), lambda h, b: (h * _NBLK + b, 0))

_k1 = pl.pallas_call(
    _k1_body,
    grid=(2, _NBLK),
    in_specs=[
        _row_spec,
        pl.BlockSpec((_F, _F), lambda h, b: (0, h)),
        pl.BlockSpec((8, _F), lambda h, b: (0, h)),
        _deg_lo,
        _deg_hi,
    ],
    out_specs=_out2n_spec,
    out_shape=jax.ShapeDtypeStruct((2 * _NP, _F), jnp.float32),
)

_k23 = pl.pallas_call(
    _k23_body,
    grid=(2, _NBLK),
    in_specs=[
        _row_lo,
        _row_hi,
        pl.BlockSpec((_F, _F), lambda h, b: (0, h)),
        pl.BlockSpec((_F, _F), lambda h, b: (1, h)),
        pl.BlockSpec((8, _F), lambda h, b: (0, h)),
        _deg_lo,
        _deg_hi,
    ],
    out_specs=_out2n_spec,
    out_shape=jax.ShapeDtypeStruct((2 * _NP, _F), jnp.float32),
)

_kout = pl.pallas_call(
    _kout_body,
    grid=(_NBLK,),
    in_specs=[
        pl.BlockSpec((_BN, _F), lambda b: (b, 0)),
        pl.BlockSpec((_BN, _F), lambda b: (_NBLK + b, 0)),
        pl.BlockSpec((_BN, _F), lambda b: (b, 0)),
        pl.BlockSpec((_BN, _F), lambda b: (_NBLK + b, 0)),
        pl.BlockSpec((_BN, _F), lambda b: (b, 0)),
        pl.BlockSpec((_BN, _F), lambda b: (_NBLK + b, 0)),
        pl.BlockSpec((6 * _F, _F), lambda b: (0, 0)),
        pl.BlockSpec((8, _F), lambda b: (0, 0)),
        pl.BlockSpec((_BN, 1), lambda b: (b, 0)),
        pl.BlockSpec((_BN, 1), lambda b: (_NBLK + b, 0)),
    ],
    out_specs=pl.BlockSpec((_BN, _F), lambda b: (b, 0)),
    out_shape=jax.ShapeDtypeStruct((_NP, _F), jnp.float32),
)


def kernel(x, edge_index, W1, b1, W2, b2, W3, b3, Wout, bout):
    src = edge_index[0]
    dst = edge_index[1]
    npad = _EPAD - _E
    fill = jnp.arange(npad, dtype=jnp.int32)
    padidx = _N + (fill % (_NP - _N))
    src_p = jnp.concatenate([src, padidx])
    dst_p = jnp.concatenate([dst, padidx])
    src_sh = jnp.concatenate([src_p, src_p + _NP])

    dd = _deg(dst_p).reshape(2 * _NB_DEG, 1)

    b1b = jnp.broadcast_to(b1, (8, _H))
    b2b = jnp.broadcast_to(b2, (8, _H))
    b3b = jnp.broadcast_to(b3, (8, _H))
    wout_p = jnp.pad(Wout, ((0, 0), (0, _F - _C)))
    bout_p = jnp.broadcast_to(jnp.pad(bout, (0, _F - _C)), (8, _F))

    x_p = jnp.pad(x, ((0, _NP - _N), (0, 0)))
    g1 = _k1(x_p, W1, b1b, dd, dd)
    s1 = _agg(g1, src_sh, dst_p)
    g2 = _k23(s1, s1, W2, W2, b2b, dd, dd)
    s2 = _agg(g2, src_sh, dst_p)
    g3 = _k23(s2, s2, W3, W3, b3b, dd, dd)
    s3 = _agg(g3, src_sh, dst_p)

    out = _kout(s1, s1, s2, s2, s3, s3, wout_p, bout_p, dd, dd)
    return out[:_N, :_C]


# speedup vs baseline: 1.0897x; 1.0134x over previous
import functools

import jax
import jax.numpy as jnp
from jax import lax
from jax.experimental import pallas as pl
from jax.experimental.pallas import tpu as pltpu
from jax.experimental.pallas import tpu_sc as plsc

_N = 10000
_E = 320000
_F = 128
_H = 256
_C = 40

_NC = 2
_NS = 16
_CHUNK = 128
_EPAD = 323584
_ECHUNKS_AGG = _EPAD // (_NS * _CHUNK)
_EPW_AGG = _EPAD // _NS
_ECHUNKS_DEG = _EPAD // (_NC * _NS * _CHUNK)
_EPW_DEG = _EPAD // (_NC * _NS)
_NP = 10240
_NR = _NP
_NB_DEG = 10240
_ROWS_PER_SUB = _NP // _NS
_INIT_CHUNK = 128

_BN = 1024
_NBLK = _NP // _BN


_sc_mesh = plsc.VectorSubcoreMesh(core_axis_name="c", subcore_axis_name="s")


@functools.partial(
    pl.kernel,
    out_type=jax.ShapeDtypeStruct((_NC * _NB_DEG,), jnp.float32),
    mesh=_sc_mesh,
    scratch_types=[
        pltpu.VMEM((4, _CHUNK), jnp.int32),
        pltpu.VMEM((128,), jnp.float32),
        pltpu.VMEM((640,), jnp.float32),
        pltpu.VMEM_SHARED((_NB_DEG,), jnp.float32),
        pltpu.SemaphoreType.DMA,
        pltpu.SemaphoreType.DMA,
        pltpu.SemaphoreType.DMA,
        pltpu.SemaphoreType.DMA,
        pltpu.SemaphoreType.DMA,
        pltpu.SemaphoreType.DMA,
        pltpu.SemaphoreType.DMA,
        pltpu.SemaphoreType.DMA,
    ],
)
def _deg(dst_hbm, out_hbm, dbs, onesb, zb, acc, i0, i1, i2, i3,
         s0, s1, s2, s3):
    c = lax.axis_index("c")
    s = lax.axis_index("s")
    w = c * _NS + s
    isem = (i0, i1, i2, i3)
    ssem = (s0, s1, s2, s3)
    ebase = w * _EPW_DEG
    ones = onesb.at[pl.ds(0, _CHUNK)]

    def iload(j, q):
        off = pl.multiple_of(ebase + j * _CHUNK, _CHUNK)
        pltpu.async_copy(dst_hbm.at[pl.ds(off, _CHUNK)], dbs.at[q], isem[q])

    iload(0, 0)
    iload(1, 1)
    for i in range(640 // 16):
        zb[pl.ds(i * 16, 16)] = jnp.zeros((16,), jnp.float32)
    for i in range(128 // 16):
        onesb[pl.ds(i * 16, 16)] = jnp.ones((16,), jnp.float32)
    pltpu.sync_copy(zb, acc.at[pl.ds(s * 640, 640)])
    plsc.subcore_barrier()

    def emit(j, q, do_swait, do_iload):
        pltpu.make_async_copy(dst_hbm.at[pl.ds(0, _CHUNK)], dbs.at[q],
                              isem[q]).wait()
        pltpu.async_copy(ones, acc.at[dbs.at[q]], ssem[q], add=True)
        if do_swait:
            q2 = (q + 2) % 4
            pltpu.make_async_copy(ones, acc.at[dbs.at[q2]], ssem[q2]).wait()
        if do_iload:
            iload(j + 2, (q + 2) % 4)

    emit(0, 0, False, True)
    emit(1, 1, False, True)
    emit(2, 2, True, True)
    emit(3, 3, True, True)

    def body(i, carry):
        j = 4 * i
        emit(j + 0, 0, True, True)
        emit(j + 1, 1, True, True)
        emit(j + 2, 2, True, True)
        emit(j + 3, 3, True, True)
        return carry

    lax.fori_loop(1, 18, body, 0)
    emit(72, 0, True, True)
    emit(73, 1, True, True)
    emit(74, 2, True, True)
    emit(75, 3, True, True)
    emit(76, 0, True, True)
    emit(77, 1, True, False)
    emit(78, 2, True, False)
    pltpu.make_async_copy(ones, acc.at[dbs.at[1]], ssem[1]).wait()
    pltpu.make_async_copy(ones, acc.at[dbs.at[2]], ssem[2]).wait()
    plsc.subcore_barrier()
    pltpu.sync_copy(acc.at[pl.ds(s * 640, 640)], zb)
    pltpu.sync_copy(zb, out_hbm.at[pl.ds(c * _NB_DEG + s * 640, 640)])


@functools.partial(
    pl.kernel,
    out_type=jax.ShapeDtypeStruct((2 * _NP, _F), jnp.float32),
    mesh=_sc_mesh,
    scratch_types=[
        pltpu.VMEM((4, _CHUNK), jnp.int32),
        pltpu.VMEM((4, _CHUNK), jnp.int32),
        pltpu.VMEM((_CHUNK, _F), jnp.float32),
        pltpu.VMEM((_CHUNK, _F), jnp.float32),
        pltpu.VMEM_SHARED((_NR, _F), jnp.float32),
        pltpu.SemaphoreType.DMA,
        pltpu.SemaphoreType.DMA,
        pltpu.SemaphoreType.DMA,
        pltpu.SemaphoreType.DMA,
        pltpu.SemaphoreType.DMA,
        pltpu.SemaphoreType.DMA,
        pltpu.SemaphoreType.DMA,
        pltpu.SemaphoreType.DMA,
    ],
)
def _agg(g_hbm, src_hbm, dst_hbm, out_hbm, sbs, dbs, rows0, rows1,
         acc, i0, i1, i2, i3, r0s, r1s, s0s, s1s):
    c = lax.axis_index("c")
    s = lax.axis_index("s")
    rbase = s * _ROWS_PER_SUB
    sbase = c * _EPAD + s * _EPW_AGG
    dbase = s * _EPW_AGG
    isem = (i0, i1, i2, i3)
    rsem = (r0s, r1s)
    ssem = (s0s, s1s)
    rows = (rows0, rows1)

    def iload(j, q):
        off = pl.multiple_of(j * _CHUNK, _CHUNK)
        pltpu.async_copy(src_hbm.at[pl.ds(sbase + off, _CHUNK)],
                         sbs.at[q], isem[q])
        pltpu.async_copy(dst_hbm.at[pl.ds(dbase + off, _CHUNK)],
                         dbs.at[q], isem[q])

    def iwait(q):
        pltpu.make_async_copy(src_hbm.at[pl.ds(0, _CHUNK)], sbs.at[q],
                              isem[q]).wait()
        pltpu.make_async_copy(dst_hbm.at[pl.ds(0, _CHUNK)], dbs.at[q],
                              isem[q]).wait()

    iload(0, 0)
    iload(1, 1)
    iload(2, 2)
    nck = _ROWS_PER_SUB // _INIT_CHUNK
    pltpu.async_copy(g_hbm.at[pl.ds(c * _NP + rbase, _INIT_CHUNK)],
                     rows0, r0s)
    pltpu.async_copy(g_hbm.at[pl.ds(c * _NP + rbase + _INIT_CHUNK,
                                    _INIT_CHUNK)], rows1, r1s)
    for k in range(nck):
        b = k % 2
        pltpu.make_async_copy(g_hbm.at[pl.ds(0, _INIT_CHUNK)], rows[b],
                              rsem[b]).wait()
        pltpu.sync_copy(rows[b], acc.at[pl.ds(rbase + k * _INIT_CHUNK,
                                              _INIT_CHUNK)])
        if k + 2 < nck:
            pltpu.async_copy(
                g_hbm.at[pl.ds(c * _NP + rbase + (k + 2) * _INIT_CHUNK,
                               _INIT_CHUNK)], rows[b], rsem[b])
    plsc.subcore_barrier()


    def emit(j, q, b, first, has_next, do_iload):
        pltpu.make_async_copy(g_hbm.at[pl.ds(0, _CHUNK)], rows[b],
                              rsem[b]).wait()
        pltpu.async_copy(rows[b], acc.at[dbs.at[q]], ssem[b], add=True)
        if has_next:
            if not first:
                pltpu.make_async_copy(rows[1 - b], acc.at[dbs.at[q]],
                                      ssem[1 - b]).wait()
            qn = (q + 1) % 4
            iwait(qn)
            pltpu.async_copy(g_hbm.at[sbs.at[qn]], rows[1 - b], rsem[1 - b])
        if do_iload:
            iload(j + 3, (q + 3) % 4)

    iwait(0)
    pltpu.async_copy(g_hbm.at[sbs.at[0]], rows0, r0s)

    emit(0, 0, 0, True, True, True)
    emit(1, 1, 1, False, True, True)
    emit(2, 2, 0, False, True, True)
    emit(3, 3, 1, False, True, True)

    def body(i, carry):
        j = 4 * i
        emit(j + 0, 0, 0, False, True, True)
        emit(j + 1, 1, 1, False, True, True)
        emit(j + 2, 2, 0, False, True, True)
        emit(j + 3, 3, 1, False, True, True)
        return carry

    lax.fori_loop(1, 38, body, 0)
    emit(152, 0, 0, False, True, True)
    emit(153, 1, 1, False, True, True)
    emit(154, 2, 0, False, True, True)
    emit(155, 3, 1, False, True, False)
    emit(156, 0, 0, False, True, False)
    emit(157, 1, 1, False, False, False)
    pltpu.make_async_copy(rows0, acc.at[dbs.at[0]], s0s).wait()
    pltpu.make_async_copy(rows1, acc.at[dbs.at[1]], s1s).wait()
    plsc.subcore_barrier()
    pltpu.async_copy(acc.at[pl.ds(rbase, _INIT_CHUNK)], rows0, r0s)
    pltpu.async_copy(acc.at[pl.ds(rbase + _INIT_CHUNK, _INIT_CHUNK)],
                     rows1, r1s)
    for k in range(nck):
        b = k % 2
        pltpu.make_async_copy(acc.at[pl.ds(rbase, _INIT_CHUNK)], rows[b],
                              rsem[b]).wait()
        pltpu.sync_copy(rows[b],
                        out_hbm.at[pl.ds(c * _NP + rbase + k * _INIT_CHUNK,
                                         _INIT_CHUNK)])
        if k + 2 < nck:
            pltpu.async_copy(
                acc.at[pl.ds(rbase + (k + 2) * _INIT_CHUNK, _INIT_CHUNK)],
                rows[b], rsem[b])


def _dinv(d0, d1):
    return lax.rsqrt(d0 + d1 + 1.0)


def _rowmask(dinv):
    rid = (lax.broadcasted_iota(jnp.int32, dinv.shape, 0)
           + pl.program_id(1) * _BN)
    return jnp.where(rid < _N, dinv, 0.0)


def _k1_body(x_ref, w_ref, b_ref, d0_ref, d1_ref, out_ref):
    dinv = _dinv(d0_ref[...], d1_ref[...])
    h = jnp.dot(x_ref[...], w_ref[...], preferred_element_type=jnp.float32)
    out_ref[...] = (h + b_ref[0:1, :]) * _rowmask(dinv)


def _k23_body(s0_ref, s1_ref, wa_ref, wb_ref, b_ref, d0_ref, d1_ref, out_ref):
    dinv = _dinv(d0_ref[...], d1_ref[...])
    xa = jax.nn.relu(dinv * s0_ref[...])
    xb = jax.nn.relu(dinv * s1_ref[...])
    h = (jnp.dot(xa, wa_ref[...], preferred_element_type=jnp.float32)
         + jnp.dot(xb, wb_ref[...], preferred_element_type=jnp.float32))
    out_ref[...] = (h + b_ref[0:1, :]) * _rowmask(dinv)


def _kout_body(s1a, s1b, s2a, s2b, s3a, s3b, w_ref, b_ref, d0_ref, d1_ref,
               out_ref):
    dinv = _dinv(d0_ref[...], d1_ref[...])
    acc = jnp.broadcast_to(b_ref[0:1, :], out_ref.shape)
    for l, sref in enumerate((s1a, s1b, s2a, s2b, s3a, s3b)):
        xl = jax.nn.relu(dinv * sref[...])
        acc = acc + jnp.dot(xl, w_ref[pl.ds(l * _F, _F), :],
                            preferred_element_type=jnp.float32)
    out_ref[...] = acc


_row_spec = pl.BlockSpec((_BN, _F), lambda h, b: (b, 0))
_row_lo = pl.BlockSpec((_BN, _F), lambda h, b: (b, 0))
_row_hi = pl.BlockSpec((_BN, _F), lambda h, b: (_NBLK + b, 0))
_deg_lo = pl.BlockSpec((_BN, 1), lambda h, b: (b, 0))
_deg_hi = pl.BlockSpec((_BN, 1), lambda h, b: (_NBLK + b, 0))
_out2n_spec = pl.BlockSpec((_BN, _F), lambda h, b: (h * _NBLK + b, 0))

_k1 = pl.pallas_call(
    _k1_body,
    grid=(2, _NBLK),
    in_specs=[
        _row_spec,
        pl.BlockSpec((_F, _F), lambda h, b: (0, h)),
        pl.BlockSpec((8, _F), lambda h, b: (0, h)),
        _deg_lo,
        _deg_hi,
    ],
    out_specs=_out2n_spec,
    out_shape=jax.ShapeDtypeStruct((2 * _NP, _F), jnp.float32),
)

_k23 = pl.pallas_call(
    _k23_body,
    grid=(2, _NBLK),
    in_specs=[
        _row_lo,
        _row_hi,
        pl.BlockSpec((_F, _F), lambda h, b: (0, h)),
        pl.BlockSpec((_F, _F), lambda h, b: (1, h)),
        pl.BlockSpec((8, _F), lambda h, b: (0, h)),
        _deg_lo,
        _deg_hi,
    ],
    out_specs=_out2n_spec,
    out_shape=jax.ShapeDtypeStruct((2 * _NP, _F), jnp.float32),
)

_kout = pl.pallas_call(
    _kout_body,
    grid=(_NBLK,),
    in_specs=[
        pl.BlockSpec((_BN, _F), lambda b: (b, 0)),
        pl.BlockSpec((_BN, _F), lambda b: (_NBLK + b, 0)),
        pl.BlockSpec((_BN, _F), lambda b: (b, 0)),
        pl.BlockSpec((_BN, _F), lambda b: (_NBLK + b, 0)),
        pl.BlockSpec((_BN, _F), lambda b: (b, 0)),
        pl.BlockSpec((_BN, _F), lambda b: (_NBLK + b, 0)),
        pl.BlockSpec((6 * _F, _F), lambda b: (0, 0)),
        pl.BlockSpec((8, _F), lambda b: (0, 0)),
        pl.BlockSpec((_BN, 1), lambda b: (b, 0)),
        pl.BlockSpec((_BN, 1), lambda b: (_NBLK + b, 0)),
    ],
    out_specs=pl.BlockSpec((_BN, _F), lambda b: (b, 0)),
    out_shape=jax.ShapeDtypeStruct((_NP, _F), jnp.float32),
)


def kernel(x, edge_index, W1, b1, W2, b2, W3, b3, Wout, bout):
    src = edge_index[0]
    dst = edge_index[1]
    npad = _EPAD - _E
    fill = jnp.arange(npad, dtype=jnp.int32)
    padidx = _N + (fill % (_NP - _N))
    src_p = jnp.concatenate([src, padidx])
    dst_p = jnp.concatenate([dst, padidx])
    src_sh = jnp.concatenate([src_p, src_p + _NP])

    dd = _deg(dst_p).reshape(2 * _NB_DEG, 1)

    b1b = jnp.broadcast_to(b1, (8, _H))
    b2b = jnp.broadcast_to(b2, (8, _H))
    b3b = jnp.broadcast_to(b3, (8, _H))
    wout_p = jnp.pad(Wout, ((0, 0), (0, _F - _C)))
    bout_p = jnp.broadcast_to(jnp.pad(bout, (0, _F - _C)), (8, _F))

    x_p = jnp.pad(x, ((0, _NP - _N), (0, 0)))
    g1 = _k1(x_p, W1, b1b, dd, dd)
    s1 = _agg(g1, src_sh, dst_p)
    g2 = _k23(s1, s1, W2, W2, b2b, dd, dd)
    s2 = _agg(g2, src_sh, dst_p)
    g3 = _k23(s2, s2, W3, W3, b3b, dd, dd)
    s3 = _agg(g3, src_sh, dst_p)

    out = _kout(s1, s1, s2, s2, s3, s3, wout_p, bout_p, dd, dd)
    return out[:_N, :_C]
